# baseline (device time: 26590 ns/iter reference)
import jax
import jax.numpy as jnp
from jax import lax
from jax.experimental import pallas as pl
from jax.experimental.pallas import tpu as pltpu

N_DEV = 4


def kernel(x):
    m, n = x.shape
    out_dtype = jnp.bfloat16

    def body(x_ref, o_ref, stats_ref, send_sems, recv_sems):
        my = lax.axis_index("i")

        barrier = pltpu.get_barrier_semaphore()
        for d in range(1, N_DEV):
            pl.semaphore_signal(
                barrier,
                inc=1,
                device_id=((my + d) % N_DEV,),
                device_id_type=pl.DeviceIdType.MESH,
            )

        xv = x_ref[:, :]
        m_loc = jnp.max(xv, axis=1)
        e = jnp.exp((xv - m_loc[:, None]).astype(jnp.bfloat16))
        s_loc = jnp.sum(e, axis=1, dtype=jnp.float32)
        stacked = jnp.stack([m_loc, s_loc])

        pl.semaphore_wait(barrier, N_DEV - 1)

        for i in range(N_DEV):

            @pl.when(my == i)
            def _(i=i, stacked=stacked):
                stats_ref[i] = stacked
                descs = []
                for d in range(1, N_DEV):
                    j = (i + d) % N_DEV
                    r = pltpu.make_async_remote_copy(
                        src_ref=stats_ref.at[i],
                        dst_ref=stats_ref.at[i],
                        send_sem=send_sems.at[j],
                        recv_sem=recv_sems.at[i],
                        device_id=(j,),
                        device_id_type=pl.DeviceIdType.MESH,
                    )
                    r.start()
                    descs.append(r)
                for r in descs:
                    r.wait_send()

        for i in range(N_DEV):

            @pl.when(my != i)
            def _(i=i):
                r = pltpu.make_async_remote_copy(
                    src_ref=stats_ref.at[i],
                    dst_ref=stats_ref.at[i],
                    send_sem=send_sems.at[i],
                    recv_sem=recv_sems.at[i],
                    device_id=(0,),
                    device_id_type=pl.DeviceIdType.MESH,
                )
                r.wait_recv()

        allm = stats_ref[:, 0, :]
        alls = stats_ref[:, 1, :]
        gmax = jnp.max(allm, axis=0)
        gsum = jnp.sum(alls * jnp.exp(allm - gmax[None, :]), axis=0)
        scale = (jnp.exp(m_loc - gmax) / gsum).astype(jnp.bfloat16)
        o_ref[:, :] = e * scale[:, None]

    return pl.pallas_call(
        body,
        out_shape=jax.ShapeDtypeStruct((m, n), out_dtype),
        in_specs=[pl.BlockSpec(memory_space=pltpu.VMEM)],
        out_specs=pl.BlockSpec(memory_space=pltpu.VMEM),
        scratch_shapes=[
            pltpu.VMEM((N_DEV, 2, m), jnp.float32),
            pltpu.SemaphoreType.DMA((N_DEV,)),
            pltpu.SemaphoreType.DMA((N_DEV,)),
        ],
        compiler_params=pltpu.CompilerParams(
            collective_id=0, vmem_limit_bytes=100 * 1024 * 1024
        ),
    )(x)


# device time: 24984 ns/iter; 1.0643x vs baseline; 1.0643x over previous
import jax
import jax.numpy as jnp
from jax import lax
from jax.experimental import pallas as pl
from jax.experimental.pallas import tpu as pltpu

N_DEV = 4
CH = 4


def kernel(x):
    m, n = x.shape
    mc = m // CH
    out_dtype = jnp.bfloat16

    def body(x_ref, o_ref, stats_ref, send_sems, recv_sems):
        my = lax.axis_index("i")

        barrier = pltpu.get_barrier_semaphore()
        for d in range(1, N_DEV):
            pl.semaphore_signal(
                barrier,
                inc=1,
                device_id=((my + d) % N_DEV,),
                device_id_type=pl.DeviceIdType.MESH,
            )

        es = []
        m_locs = []
        send_descs = []
        for c in range(CH):
            xc = x_ref[c * mc : (c + 1) * mc, :]
            m_c = jnp.max(xc, axis=1)
            e_c = jnp.exp(xc - m_c[:, None])
            s_c = jnp.sum(e_c, axis=1)
            es.append(e_c)
            m_locs.append(m_c)
            stacked = jnp.stack([m_c, s_c])

            if c == 0:
                pl.semaphore_wait(barrier, N_DEV - 1)

            for i in range(N_DEV):

                @pl.when(my == i)
                def _(i=i, c=c, stacked=stacked):
                    stats_ref[c, i] = stacked
                    for d in range(1, N_DEV):
                        j = (i + d) % N_DEV
                        r = pltpu.make_async_remote_copy(
                            src_ref=stats_ref.at[c, i],
                            dst_ref=stats_ref.at[c, i],
                            send_sem=send_sems.at[c, j],
                            recv_sem=recv_sems.at[c, i],
                            device_id=(j,),
                            device_id_type=pl.DeviceIdType.MESH,
                        )
                        r.start()

        for c in range(CH):
            for i in range(N_DEV):

                @pl.when(my != i)
                def _(i=i, c=c):
                    r = pltpu.make_async_remote_copy(
                        src_ref=stats_ref.at[c, i],
                        dst_ref=stats_ref.at[c, i],
                        send_sem=send_sems.at[c, i],
                        recv_sem=recv_sems.at[c, i],
                        device_id=(0,),
                        device_id_type=pl.DeviceIdType.MESH,
                    )
                    r.wait_recv()

            allm = stats_ref[c, :, 0, :]
            alls = stats_ref[c, :, 1, :]
            gmax = jnp.max(allm, axis=0)
            gsum = jnp.sum(alls * jnp.exp(allm - gmax[None, :]), axis=0)
            scale = jnp.exp(m_locs[c] - gmax) / gsum
            o_ref[c * mc : (c + 1) * mc, :] = (
                es[c] * scale[:, None]
            ).astype(out_dtype)

        for c in range(CH):
            for i in range(N_DEV):

                @pl.when(my != i)
                def _(i=i, c=c):
                    r = pltpu.make_async_remote_copy(
                        src_ref=stats_ref.at[c, 0],
                        dst_ref=stats_ref.at[c, 0],
                        send_sem=send_sems.at[c, i],
                        recv_sem=recv_sems.at[c, i],
                        device_id=(0,),
                        device_id_type=pl.DeviceIdType.MESH,
                    )
                    r.wait_send()

    return pl.pallas_call(
        body,
        out_shape=jax.ShapeDtypeStruct((m, n), out_dtype),
        in_specs=[pl.BlockSpec(memory_space=pltpu.VMEM)],
        out_specs=pl.BlockSpec(memory_space=pltpu.VMEM),
        scratch_shapes=[
            pltpu.VMEM((CH, N_DEV, 2, mc), jnp.float32),
            pltpu.SemaphoreType.DMA((CH, N_DEV)),
            pltpu.SemaphoreType.DMA((CH, N_DEV)),
        ],
        compiler_params=pltpu.CompilerParams(
            collective_id=0, vmem_limit_bytes=100 * 1024 * 1024
        ),
    )(x)
